# baseline (speedup 1.0000x reference)
import jax
import jax.numpy as jnp
from jax import lax
from jax.experimental import pallas as pl
from jax.experimental.pallas import tpu as pltpu
from jax.experimental.pallas import tpu_sc as plsc

DENSE = 16384
B = 4096
K = 64
NC = 2
NS = 16
L = 16
NW = NC * NS
ROWS_PER_W = B // NW
KCHUNKS = K // L
RPB = 2
NPAIR = ROWS_PER_W // RPB


def _sc_body(ids_hbm, vals_hbm, zeros_hbm, out_hbm, ids_v, vals_v, buf0,
             buf1, sem0, sem1):
    wid = lax.axis_index("s") * NC + lax.axis_index("c")
    base = wid * ROWS_PER_W

    z0 = pltpu.async_copy(zeros_hbm, buf0, sem0)
    z1 = pltpu.async_copy(zeros_hbm, buf1, sem1)
    pltpu.sync_copy(ids_hbm.at[pl.ds(base, ROWS_PER_W)], ids_v)
    pltpu.sync_copy(vals_hbm.at[pl.ds(base, ROWS_PER_W)], vals_v)
    z0.wait()
    z1.wait()

    zeros_f = jnp.zeros((L,), jnp.float32)
    rowsel = tuple(jnp.full((L,), rr, jnp.int32) for rr in range(RPB))

    def scatter_add_pair(buf, pair):
        for rr in range(RPB):
            row = RPB * pair + rr
            for c in range(KCHUNKS):
                idx = ids_v[row, pl.ds(c * L, L)]
                v = vals_v[row, pl.ds(c * L, L)]
                plsc.addupdate_scatter(buf, [rowsel[rr], idx], v)

    def scatter_zero_pair(buf, pair):
        for rr in range(RPB):
            row = RPB * pair + rr
            for c in range(KCHUNKS):
                idx = ids_v[row, pl.ds(c * L, L)]
                plsc.store_scatter(buf, [rowsel[rr], idx], zeros_f)

    bufs = (buf0, buf1)
    sems = (sem0, sem1)

    for b in range(2):
        scatter_add_pair(bufs[b], b)
        pltpu.async_copy(bufs[b], out_hbm.at[pl.ds(base + RPB * b, RPB)],
                         sems[b])

    def step(i, _):
        for b in range(2):
            pair = 2 * i + b
            pltpu.make_async_copy(
                bufs[b], out_hbm.at[pl.ds(base, RPB)], sems[b]).wait()
            scatter_zero_pair(bufs[b], pair - 2)
            scatter_add_pair(bufs[b], pair)
            pltpu.async_copy(
                bufs[b], out_hbm.at[pl.ds(base + RPB * pair, RPB)], sems[b])
        return 0

    lax.fori_loop(1, NPAIR // 2, step, 0, unroll=False)

    for b in range(2):
        pltpu.make_async_copy(
            bufs[b], out_hbm.at[pl.ds(base, RPB)], sems[b]).wait()


@jax.jit
def _sparse2dense(spike_ids, spike_vals):
    zeros_blk = jnp.zeros((RPB, DENSE), jnp.float32)
    mesh = plsc.VectorSubcoreMesh(
        core_axis_name="c", subcore_axis_name="s",
        num_cores=NC, num_subcores=NS)
    return pl.kernel(
        _sc_body,
        out_type=jax.ShapeDtypeStruct((B, DENSE), jnp.float32),
        mesh=mesh,
        compiler_params=pltpu.CompilerParams(needs_layout_passes=False),
        scratch_types=[
            pltpu.VMEM((ROWS_PER_W, K), jnp.int32),
            pltpu.VMEM((ROWS_PER_W, K), jnp.float32),
            pltpu.VMEM((RPB, DENSE), jnp.float32),
            pltpu.VMEM((RPB, DENSE), jnp.float32),
            pltpu.SemaphoreType.DMA,
            pltpu.SemaphoreType.DMA,
        ],
    )(spike_ids, spike_vals, zeros_blk)


def kernel(spike_ids, spike_vals):
    return _sparse2dense(spike_ids, spike_vals)


# speedup vs baseline: 1.1007x; 1.1007x over previous
import jax
import jax.numpy as jnp
from jax import lax
from jax.experimental import pallas as pl
from jax.experimental.pallas import tpu as pltpu
from jax.experimental.pallas import tpu_sc as plsc

DENSE = 16384
B = 4096
K = 64
NC = 2
NS = 16
L = 16
NW = NC * NS
ROWS_PER_W = B // NW
KCHUNKS = K // L
RPB = 2
NPAIR = ROWS_PER_W // RPB


def _sc_body(ids_hbm, vals_hbm, out_hbm, ids_v, vals_v, buf0,
             buf1, sem0, sem1):
    wid = lax.axis_index("s") * NC + lax.axis_index("c")
    base = wid * ROWS_PER_W

    st0 = pltpu.async_copy(ids_hbm.at[pl.ds(base, ROWS_PER_W)], ids_v, sem0)
    st1 = pltpu.async_copy(vals_hbm.at[pl.ds(base, ROWS_PER_W)], vals_v,
                           sem1)

    zeros_f = jnp.zeros((L,), jnp.float32)
    rowsel = tuple(jnp.full((L,), rr, jnp.int32) for rr in range(RPB))

    def _zero(j, _):
        for rr in range(RPB):
            buf0[rr, pl.ds(j * L, L)] = zeros_f
            buf1[rr, pl.ds(j * L, L)] = zeros_f
        return 0
    lax.fori_loop(0, DENSE // L, _zero, 0, unroll=8)
    st0.wait()
    st1.wait()

    def scatter_add_pair(buf, pair):
        for rr in range(RPB):
            row = RPB * pair + rr
            for c in range(KCHUNKS):
                idx = ids_v[row, pl.ds(c * L, L)]
                v = vals_v[row, pl.ds(c * L, L)]
                plsc.addupdate_scatter(buf, [rowsel[rr], idx], v)

    def scatter_zero_pair(buf, pair):
        for rr in range(RPB):
            row = RPB * pair + rr
            for c in range(KCHUNKS):
                idx = ids_v[row, pl.ds(c * L, L)]
                plsc.store_scatter(buf, [rowsel[rr], idx], zeros_f)

    bufs = (buf0, buf1)
    sems = (sem0, sem1)

    for b in range(2):
        scatter_add_pair(bufs[b], b)
        pltpu.async_copy(bufs[b], out_hbm.at[pl.ds(base + RPB * b, RPB)],
                         sems[b])

    def step(i, _):
        for b in range(2):
            pair = 2 * i + b
            pltpu.make_async_copy(
                bufs[b], out_hbm.at[pl.ds(base, RPB)], sems[b]).wait()
            scatter_zero_pair(bufs[b], pair - 2)
            scatter_add_pair(bufs[b], pair)
            pltpu.async_copy(
                bufs[b], out_hbm.at[pl.ds(base + RPB * pair, RPB)], sems[b])
        return 0

    lax.fori_loop(1, NPAIR // 2, step, 0, unroll=False)

    for b in range(2):
        pltpu.make_async_copy(
            bufs[b], out_hbm.at[pl.ds(base, RPB)], sems[b]).wait()


@jax.jit
def _sparse2dense(spike_ids, spike_vals):
    mesh = plsc.VectorSubcoreMesh(
        core_axis_name="c", subcore_axis_name="s",
        num_cores=NC, num_subcores=NS)
    return pl.kernel(
        _sc_body,
        out_type=jax.ShapeDtypeStruct((B, DENSE), jnp.float32),
        mesh=mesh,
        compiler_params=pltpu.CompilerParams(needs_layout_passes=False),
        scratch_types=[
            pltpu.VMEM((ROWS_PER_W, K), jnp.int32),
            pltpu.VMEM((ROWS_PER_W, K), jnp.float32),
            pltpu.VMEM((RPB, DENSE), jnp.float32),
            pltpu.VMEM((RPB, DENSE), jnp.float32),
            pltpu.SemaphoreType.DMA,
            pltpu.SemaphoreType.DMA,
        ],
    )(spike_ids, spike_vals)


def kernel(spike_ids, spike_vals):
    return _sparse2dense(spike_ids, spike_vals)
